# baseline (speedup 1.0000x reference)
import jax
import jax.numpy as jnp
from jax.experimental import pallas as pl


def _add_kernel(blk, x_ref, pos_ref, o_ref):
    i = pl.program_id(0)
    o_ref[...] = x_ref[...] + pos_ref[pl.ds(i * blk, blk), :][:, :, None]


def kernel(x, pos_emb):
    batch, seq_len, dim = x.shape
    pos = pos_emb[:seq_len]
    xt = jnp.transpose(x, (1, 2, 0))
    blk = 4
    grid = (seq_len // blk,)
    out = pl.pallas_call(
        lambda *refs: _add_kernel(blk, *refs),
        grid=grid,
        in_specs=[
            pl.BlockSpec((blk, dim, batch), lambda i: (i, 0, 0)),
            pl.BlockSpec((seq_len, dim), lambda i: (0, 0)),
        ],
        out_specs=pl.BlockSpec((blk, dim, batch), lambda i: (i, 0, 0)),
        out_shape=jax.ShapeDtypeStruct((seq_len, dim, batch), x.dtype),
    )(xt, pos)
    return jnp.transpose(out, (2, 0, 1))


# speedup vs baseline: 1.0101x; 1.0101x over previous
import jax
import jax.numpy as jnp
from jax.experimental import pallas as pl
from jax.experimental.pallas import tpu as pltpu


def _add_kernel(blk, x_ref, post_ref, o_ref, pos_scratch):
    i = pl.program_id(0)

    @pl.when(i == 0)
    def _():
        pos_scratch[...] = jnp.swapaxes(post_ref[...], 0, 1)

    pos = pos_scratch[pl.ds(i * blk, blk), :]
    o_ref[...] = x_ref[...] + pos[:, :, None]


def kernel(x, pos_emb):
    batch, seq_len, dim = x.shape
    xt = jnp.transpose(x, (1, 2, 0))
    post = jnp.transpose(pos_emb[:seq_len])
    blk = 4
    grid = (seq_len // blk,)
    out = pl.pallas_call(
        lambda *refs: _add_kernel(blk, *refs),
        grid=grid,
        in_specs=[
            pl.BlockSpec((blk, dim, batch), lambda i: (i, 0, 0)),
            pl.BlockSpec((dim, seq_len), lambda i: (0, 0)),
        ],
        out_specs=pl.BlockSpec((blk, dim, batch), lambda i: (i, 0, 0)),
        out_shape=jax.ShapeDtypeStruct((seq_len, dim, batch), x.dtype),
        scratch_shapes=[pltpu.VMEM((seq_len, dim), x.dtype)],
    )(xt, post)
    return jnp.transpose(out, (2, 0, 1))
